# baseline (device time: 52059 ns/iter reference)
import jax
import jax.numpy as jnp
from jax import lax
from jax.experimental import pallas as pl
from jax.experimental.pallas import tpu as pltpu

N_DEV = 4
SQ = 256
SKV_SHARD = 4096
HQ = 8
DH = 128
DM = HQ * DH
SCALE = 0.08838834764831843
BLK = 64
NBLK = SKV_SHARD // BLK
NCLS = 22
CLEN = NCLS * BLK
QBLKS = SQ // BLK
LCOLS = 16
PLANE = DH + LCOLS

LAG1 = 2
LAG2 = 4


def kernel(x, Wq, K_ext, V_ext, Wo):
    x2 = x.reshape(SQ, DM)
    K3 = K_ext.reshape(SKV_SHARD, HQ, DH)
    V3 = V_ext.reshape(SKV_SHARD, HQ, DH)

    def body(x_ref, wq_ref, k_any, v_any, wo_ref, out_ref,
             k_buf, v_buf, slot_ref, recv0_ref, recv1_ref,
             kv_sems, s0_send, s0_recv, s1_send, s1_recv):
        my = lax.axis_index("i")
        p0 = my ^ 1
        p1 = 3 - my

        barrier = pltpu.get_barrier_semaphore()
        for nbr in (p0, p1):
            pl.semaphore_signal(
                barrier, inc=1,
                device_id=(nbr,), device_id_type=pl.DeviceIdType.MESH,
            )
        pl.semaphore_wait(barrier, 2)

        rb = [(3 - ((b + my) % 3)) % 3 for b in range(3)]
        climit = [jnp.where(rb[b] == 0, CLEN, CLEN - BLK) for b in range(3)]

        def kv_dmas(h, slot):
            dmas = []
            for b in range(3):
                for i in range(NCLS):
                    c = jnp.minimum(rb[b] + 3 * i, NBLK - 1)
                    row = pl.ds(c * BLK, BLK)
                    dst = pl.ds(i * BLK, BLK)
                    dmas.append(pltpu.make_async_copy(
                        k_any.at[row, h, :], k_buf.at[slot, b, dst, :],
                        kv_sems.at[slot, 0]))
                    dmas.append(pltpu.make_async_copy(
                        v_any.at[row, h, :], v_buf.at[slot, b, dst, :],
                        kv_sems.at[slot, 1]))
            return dmas

        def exchange(stage, h):
            src, dst, ssem, rsem, p = (
                (slot_ref, recv0_ref, s0_send, s0_recv, p0) if stage == 0
                else (slot_ref, recv1_ref, s1_send, s1_recv, p1))
            return pltpu.make_async_remote_copy(
                src_ref=src.at[h],
                dst_ref=dst.at[h],
                send_sem=ssem.at[h],
                recv_sem=rsem.at[h],
                device_id=(p,),
                device_id_type=pl.DeviceIdType.MESH,
            )

        pend = {0: kv_dmas(0, 0)}
        for d in pend[0]:
            d.start()

        q = jnp.dot(x_ref[...], wq_ref[...],
                    preferred_element_type=jnp.float32) * SCALE

        cols = lax.broadcasted_iota(jnp.int32, (BLK, CLEN), 1)

        s0 = {}
        s1 = {}

        def do_stage0_add_and_stage1(g):
            s0[g].wait()
            slot_ref[g] = slot_ref[g] + recv0_ref[g]
            s1[g] = exchange(1, g)
            s1[g].start()

        def do_stage1_add_and_project(f):
            s1[f].wait()
            total = slot_ref[f] + recv1_ref[f]
            ctx_n = total[:, :DH] / total[:, DH:DH + 1]
            term = jnp.dot(ctx_n, wo_ref[f * DH:(f + 1) * DH, :],
                           preferred_element_type=jnp.float32)
            if f == 0:
                out_ref[...] = term
            else:
                out_ref[...] = out_ref[...] + term

        for h in range(HQ):
            slot = h % 2
            if h + 1 < HQ:
                pend[h + 1] = kv_dmas(h + 1, (h + 1) % 2)
                for d in pend[h + 1]:
                    d.start()
            for d in pend[h]:
                d.wait()
            for qb in range(QBLKS):
                b = qb % 3
                rows = pl.ds(qb * BLK, BLK)
                q_blk = q[qb * BLK:(qb + 1) * BLK, h * DH:(h + 1) * DH]
                s = lax.dot_general(q_blk, k_buf[slot, b],
                                    (((1,), (1,)), ((), ())),
                                    preferred_element_type=jnp.float32)
                w = jnp.where(cols < climit[b], jnp.exp(s), 0.0)
                l_qb = jnp.sum(w, axis=1, keepdims=True)
                ctx_qb = jnp.dot(w, v_buf[slot, b],
                                 preferred_element_type=jnp.float32)
                slot_ref[h, rows, :DH] = ctx_qb
                slot_ref[h, rows, DH:] = jnp.broadcast_to(l_qb, (BLK, LCOLS))

                if qb in (1, 2):
                    bx = (3 - qb) % 3
                    @pl.when(my == 0)
                    def _(q_blk=q_blk, rows=rows, h=h, slot=slot, bx=bx):
                        acc_c = jnp.zeros((BLK, DH), jnp.float32)
                        acc_l = jnp.zeros((BLK, 1), jnp.float32)
                        for bb in (0, bx):
                            kx = k_buf[slot, bb, :BLK, :]
                            vx = v_buf[slot, bb, :BLK, :]
                            se = lax.dot_general(
                                q_blk, kx, (((1,), (1,)), ((), ())),
                                preferred_element_type=jnp.float32)
                            we = jnp.exp(se)
                            acc_l = acc_l + jnp.sum(we, axis=1, keepdims=True)
                            acc_c = acc_c + jnp.dot(
                                we, vx, preferred_element_type=jnp.float32)
                        slot_ref[h, rows, :DH] = (
                            slot_ref[h, rows, :DH] + acc_c)
                        slot_ref[h, rows, DH:] = (
                            slot_ref[h, rows, DH:]
                            + jnp.broadcast_to(acc_l, (BLK, LCOLS)))

            s0[h] = exchange(0, h)
            s0[h].start()
            if h >= LAG1:
                do_stage0_add_and_stage1(h - LAG1)
            if h >= LAG2:
                do_stage1_add_and_project(h - LAG2)

        for g in range(HQ - LAG1, HQ):
            do_stage0_add_and_stage1(g)
        for f in range(HQ - LAG2, HQ):
            do_stage1_add_and_project(f)

    out = pl.pallas_call(
        body,
        out_shape=jax.ShapeDtypeStruct((SQ, DM), jnp.float32),
        in_specs=[
            pl.BlockSpec(memory_space=pltpu.VMEM),
            pl.BlockSpec(memory_space=pltpu.VMEM),
            pl.BlockSpec(memory_space=pl.ANY),
            pl.BlockSpec(memory_space=pl.ANY),
            pl.BlockSpec(memory_space=pltpu.VMEM),
        ],
        out_specs=pl.BlockSpec(memory_space=pltpu.VMEM),
        scratch_shapes=[
            pltpu.VMEM((2, 3, CLEN, DH), jnp.float32),
            pltpu.VMEM((2, 3, CLEN, DH), jnp.float32),
            pltpu.VMEM((HQ, SQ, PLANE), jnp.float32),
            pltpu.VMEM((HQ, SQ, PLANE), jnp.float32),
            pltpu.VMEM((HQ, SQ, PLANE), jnp.float32),
            pltpu.SemaphoreType.DMA((2, 2)),
            pltpu.SemaphoreType.DMA((HQ,)),
            pltpu.SemaphoreType.DMA((HQ,)),
            pltpu.SemaphoreType.DMA((HQ,)),
            pltpu.SemaphoreType.DMA((HQ,)),
        ],
        compiler_params=pltpu.CompilerParams(
            collective_id=0,
            vmem_limit_bytes=100 * 1024 * 1024,
        ),
    )(x2, Wq, K3, V3, Wo)
    return out.reshape(1, SQ, DM)
